# baseline (speedup 1.0000x reference)
import functools

import jax
import jax.numpy as jnp
from jax import lax
from jax.experimental import pallas as pl
from jax.experimental.pallas import tpu as pltpu
from jax.experimental.pallas import tpu_sc as plsc

T = 2048
H = 1024
F = 2048
E = 8
SOFTCAP = 30.0

TM = 256
NT = 24
P = NT * TM
HI = H // 2
NW = 32
TPW = T // NW


def _dispatch_body(x_ref, wg_ref, p0_ref, p1_ref, w0_ref, w1_ref, te_ref):
    x = x_ref[...]
    logits = lax.dot_general(x, wg_ref[...], (((1,), (1,)), ((), ())),
                             preferred_element_type=jnp.float32)
    logits = SOFTCAP * jnp.tanh(logits / SOFTCAP)
    m = jnp.max(logits, axis=-1, keepdims=True)
    p = jnp.exp(logits - m)
    p = p / jnp.sum(p, axis=-1, keepdims=True)
    e_iota = lax.broadcasted_iota(jnp.int32, (T, E), 1)
    m1 = jnp.max(p, axis=-1, keepdims=True)
    i1 = jnp.min(jnp.where(p == m1, e_iota, E), axis=-1, keepdims=True)
    sel1 = e_iota == i1
    p2 = jnp.where(sel1, -jnp.inf, p)
    m2 = jnp.max(p2, axis=-1, keepdims=True)
    i2 = jnp.min(jnp.where(p2 == m2, e_iota, E), axis=-1, keepdims=True)
    sel2 = e_iota == i2
    s01 = (sel1 | sel2).astype(jnp.float32)

    nblk = T // TM
    lt = (lax.broadcasted_iota(jnp.int32, (TM, TM), 0) >
          lax.broadcasted_iota(jnp.int32, (TM, TM), 1)).astype(jnp.float32)
    ranks, bsums = [], []
    for b in range(nblk):
        sb = s01[b * TM:(b + 1) * TM, :]
        ranks.append(lax.dot_general(lt, sb, (((1,), (0,)), ((), ())),
                                     preferred_element_type=jnp.float32))
        bsums.append(jnp.sum(sb, axis=0, keepdims=True))
    bs = jnp.concatenate(bsums, axis=0)
    lt8 = (lax.broadcasted_iota(jnp.int32, (nblk, nblk), 0) >
           lax.broadcasted_iota(jnp.int32, (nblk, nblk), 1)).astype(jnp.float32)
    boffs = lax.dot_general(lt8, bs, (((1,), (0,)), ((), ())),
                            preferred_element_type=jnp.float32)
    rank = jnp.concatenate(
        [ranks[b] + boffs[b:b + 1, :] for b in range(nblk)], axis=0)
    counts = boffs[nblk - 1:nblk, :] + bs[nblk - 1:nblk, :]
    c_i = counts.astype(jnp.int32)
    nt_e = (c_i + (TM - 1)) // TM
    m8 = (lax.broadcasted_iota(jnp.int32, (E, E), 0) <
          lax.broadcasted_iota(jnp.int32, (E, E), 1)).astype(jnp.float32)
    offs_t = lax.dot_general(nt_e.astype(jnp.float32), m8,
                             (((1,), (0,)), ((), ())),
                             preferred_element_type=jnp.float32)
    offs_i = offs_t.astype(jnp.int32)
    dest = offs_i * TM + rank.astype(jnp.int32)
    p0_ref[...] = jnp.min(jnp.where(sel1, dest, P), axis=-1, keepdims=True)
    p1_ref[...] = jnp.min(jnp.where(sel2, dest, P), axis=-1, keepdims=True)
    w0_ref[...] = jnp.broadcast_to(m1, (T, 16))
    w1_ref[...] = jnp.broadcast_to(m2, (T, 16))
    jt = lax.broadcasted_iota(jnp.int32, (NT, E), 0)
    step = (jt >= offs_i).astype(jnp.float32)
    te = lax.dot_general(step, jnp.ones((E, 1), jnp.float32),
                         (((1,), (0,)), ((), ())),
                         preferred_element_type=jnp.float32) - 1.0
    te_ref[...] = te.astype(jnp.int32)


def _sc_scatter_body(x_hbm, p0_hbm, p1_hbm, xs_hbm, xbuf, idx0, idx1, sem):
    c = lax.axis_index("c")
    s = lax.axis_index("s")
    wid = s * 2 + c
    t0 = wid * TPW
    pltpu.sync_copy(x_hbm.at[pl.ds(t0, TPW)], xbuf)
    pltpu.sync_copy(p0_hbm.at[pl.ds(t0, TPW)], idx0)
    pltpu.sync_copy(p1_hbm.at[pl.ds(t0, TPW)], idx1)
    pltpu.async_copy(xbuf, xs_hbm.at[idx0], sem).wait()
    pltpu.async_copy(xbuf, xs_hbm.at[idx1], sem).wait()


def _ffn_body(te_ref, xs_ref, w1_ref, w3_ref, w2_ref, ys_ref):
    x = xs_ref[...]
    h1 = lax.dot_general(x, w1_ref[0], (((1,), (1,)), ((), ())),
                         preferred_element_type=jnp.float32)
    h3 = lax.dot_general(x, w3_ref[0], (((1,), (1,)), ((), ())),
                         preferred_element_type=jnp.float32)
    act = (0.5 * h1) * (1.0 + lax.erf(h1 * 0.7071067811865476)) * h3
    ys_ref[...] = lax.dot_general(act.astype(jnp.bfloat16), w2_ref[0],
                                  (((1,), (1,)), ((), ())),
                                  preferred_element_type=jnp.float32)


def _sc_combine_body(ys_hbm, p0_hbm, p1_hbm, w0_hbm, w1_hbm, y_hbm,
                     wbuf0, wbuf1, idxa, idxb, buf0, buf1, ybuf, sem0, sem1):
    c = lax.axis_index("c")
    s = lax.axis_index("s")
    wid = s * 2 + c
    t0 = wid * TPW
    pltpu.sync_copy(w0_hbm.at[pl.ds(t0, TPW)], wbuf0)
    pltpu.sync_copy(w1_hbm.at[pl.ds(t0, TPW)], wbuf1)
    for ch in range(2):
        pltpu.sync_copy(p0_hbm.at[pl.ds(t0 + ch * 32, 32)], idxa)
        pltpu.sync_copy(p1_hbm.at[pl.ds(t0 + ch * 32, 32)], idxb)
        cpa = pltpu.async_copy(ys_hbm.at[idxa], buf0, sem0)
        cpb = pltpu.async_copy(ys_hbm.at[idxb], buf1, sem1)
        cpa.wait()
        cpb.wait()

        def row_body(i, carry, ch=ch):
            wa = wbuf0[ch * 32 + i]
            wb = wbuf1[ch * 32 + i]
            for cc in range(H // 16):
                sl = pl.ds(cc * 16, 16)
                ybuf[i, sl] = wa * buf0[i, sl] + wb * buf1[i, sl]
            return carry

        lax.fori_loop(0, 32, row_body, 0)
        pltpu.sync_copy(ybuf, y_hbm.at[pl.ds(t0 + ch * 32, 32)])


@jax.jit
def kernel(hidden_states, w_gate, w1, w3, w2):
    pos0, pos1, wv0, wv1, te_col = pl.pallas_call(
        _dispatch_body,
        out_shape=(jax.ShapeDtypeStruct((T, 1), jnp.int32),
                   jax.ShapeDtypeStruct((T, 1), jnp.int32),
                   jax.ShapeDtypeStruct((T, 16), jnp.float32),
                   jax.ShapeDtypeStruct((T, 16), jnp.float32),
                   jax.ShapeDtypeStruct((NT, 1), jnp.int32)),
        in_specs=[pl.BlockSpec((T, H), lambda: (0, 0)),
                  pl.BlockSpec((E, H), lambda: (0, 0))],
        out_specs=(pl.BlockSpec((T, 1), lambda: (0, 0)),
                   pl.BlockSpec((T, 1), lambda: (0, 0)),
                   pl.BlockSpec((T, 16), lambda: (0, 0)),
                   pl.BlockSpec((T, 16), lambda: (0, 0)),
                   pl.BlockSpec((NT, 1), lambda: (0, 0))),
    )(hidden_states, w_gate)
    p0f = pos0.reshape(T)
    p1f = pos1.reshape(T)

    x_bf = hidden_states.astype(jnp.bfloat16)
    x_i32 = lax.bitcast_convert_type(x_bf.reshape(T, HI, 2), jnp.int32)

    mesh = plsc.VectorSubcoreMesh(core_axis_name="c", subcore_axis_name="s")
    xs_i32 = pl.kernel(
        _sc_scatter_body,
        out_type=jax.ShapeDtypeStruct((P, HI), jnp.int32),
        mesh=mesh,
        scratch_types=[pltpu.VMEM((TPW, HI), jnp.int32),
                       pltpu.VMEM((TPW,), jnp.int32),
                       pltpu.VMEM((TPW,), jnp.int32),
                       pltpu.SemaphoreType.DMA],
    )(x_i32, p0f, p1f)
    xs_bf = lax.bitcast_convert_type(xs_i32, jnp.bfloat16).reshape(P, H)

    w1_bf = w1.astype(jnp.bfloat16)
    w3_bf = w3.astype(jnp.bfloat16)
    w2_bf = w2.astype(jnp.bfloat16)
    te_arr = te_col.reshape(NT)

    grid_spec = pltpu.PrefetchScalarGridSpec(
        num_scalar_prefetch=1,
        grid=(NT,),
        in_specs=[
            pl.BlockSpec((TM, H), lambda j, te: (j, 0)),
            pl.BlockSpec((1, F, H), lambda j, te: (te[j], 0, 0)),
            pl.BlockSpec((1, F, H), lambda j, te: (te[j], 0, 0)),
            pl.BlockSpec((1, H, F), lambda j, te: (te[j], 0, 0)),
        ],
        out_specs=pl.BlockSpec((TM, H), lambda j, te: (j, 0)),
    )
    ys = pl.pallas_call(
        _ffn_body,
        grid_spec=grid_spec,
        out_shape=jax.ShapeDtypeStruct((P, H), jnp.float32),
        compiler_params=pltpu.CompilerParams(
            dimension_semantics=("arbitrary",)),
    )(te_arr, xs_bf, w1_bf, w3_bf, w2_bf)

    y = pl.kernel(
        _sc_combine_body,
        out_type=jax.ShapeDtypeStruct((T, H), jnp.float32),
        mesh=mesh,
        scratch_types=[pltpu.VMEM((TPW, 16), jnp.float32),
                       pltpu.VMEM((TPW, 16), jnp.float32),
                       pltpu.VMEM((32,), jnp.int32),
                       pltpu.VMEM((32,), jnp.int32),
                       pltpu.VMEM((32, H), jnp.float32),
                       pltpu.VMEM((32, H), jnp.float32),
                       pltpu.VMEM((32, H), jnp.float32),
                       pltpu.SemaphoreType.DMA,
                       pltpu.SemaphoreType.DMA],
    )(ys, p0f, p1f, wv0, wv1)
    return y


# speedup vs baseline: 1.6618x; 1.6618x over previous
import functools

import jax
import jax.numpy as jnp
from jax import lax
from jax.experimental import pallas as pl
from jax.experimental.pallas import tpu as pltpu
from jax.experimental.pallas import tpu_sc as plsc

T = 2048
H = 1024
F = 2048
E = 8
SOFTCAP = 30.0

TM = 256
NT = 24
P = NT * TM
HI = H // 2
NW = 32
TPW = T // NW


def _dispatch_body(x_ref, wg_ref, p0_ref, p1_ref, w0_ref, w1_ref, te_ref):
    x = x_ref[...]
    logits = lax.dot_general(x, wg_ref[...], (((1,), (1,)), ((), ())),
                             preferred_element_type=jnp.float32)
    logits = SOFTCAP * jnp.tanh(logits / SOFTCAP)
    m = jnp.max(logits, axis=-1, keepdims=True)
    p = jnp.exp(logits - m)
    p = p / jnp.sum(p, axis=-1, keepdims=True)
    e_iota = lax.broadcasted_iota(jnp.int32, (T, E), 1)
    m1 = jnp.max(p, axis=-1, keepdims=True)
    i1 = jnp.min(jnp.where(p == m1, e_iota, E), axis=-1, keepdims=True)
    sel1 = e_iota == i1
    p2 = jnp.where(sel1, -jnp.inf, p)
    m2 = jnp.max(p2, axis=-1, keepdims=True)
    i2 = jnp.min(jnp.where(p2 == m2, e_iota, E), axis=-1, keepdims=True)
    sel2 = e_iota == i2
    s01 = (sel1 | sel2).astype(jnp.float32)

    nblk = T // TM
    lt = (lax.broadcasted_iota(jnp.int32, (TM, TM), 0) >
          lax.broadcasted_iota(jnp.int32, (TM, TM), 1)).astype(jnp.float32)
    ranks, bsums = [], []
    for b in range(nblk):
        sb = s01[b * TM:(b + 1) * TM, :]
        ranks.append(lax.dot_general(lt, sb, (((1,), (0,)), ((), ())),
                                     preferred_element_type=jnp.float32))
        bsums.append(jnp.sum(sb, axis=0, keepdims=True))
    bs = jnp.concatenate(bsums, axis=0)
    lt8 = (lax.broadcasted_iota(jnp.int32, (nblk, nblk), 0) >
           lax.broadcasted_iota(jnp.int32, (nblk, nblk), 1)).astype(jnp.float32)
    boffs = lax.dot_general(lt8, bs, (((1,), (0,)), ((), ())),
                            preferred_element_type=jnp.float32)
    rank = jnp.concatenate(
        [ranks[b] + boffs[b:b + 1, :] for b in range(nblk)], axis=0)
    counts = boffs[nblk - 1:nblk, :] + bs[nblk - 1:nblk, :]
    c_i = counts.astype(jnp.int32)
    nt_e = (c_i + (TM - 1)) // TM
    m8 = (lax.broadcasted_iota(jnp.int32, (E, E), 0) <
          lax.broadcasted_iota(jnp.int32, (E, E), 1)).astype(jnp.float32)
    offs_t = lax.dot_general(nt_e.astype(jnp.float32), m8,
                             (((1,), (0,)), ((), ())),
                             preferred_element_type=jnp.float32)
    offs_i = offs_t.astype(jnp.int32)
    dest = offs_i * TM + rank.astype(jnp.int32)
    p0_ref[...] = jnp.min(jnp.where(sel1, dest, P), axis=-1, keepdims=True)
    p1_ref[...] = jnp.min(jnp.where(sel2, dest, P), axis=-1, keepdims=True)
    w0_ref[...] = jnp.broadcast_to(m1, (T, 16))
    w1_ref[...] = jnp.broadcast_to(m2, (T, 16))
    jt = lax.broadcasted_iota(jnp.int32, (NT, E), 0)
    step = (jt >= offs_i).astype(jnp.float32)
    te = lax.dot_general(step, jnp.ones((E, 1), jnp.float32),
                         (((1,), (0,)), ((), ())),
                         preferred_element_type=jnp.float32) - 1.0
    te_ref[...] = te.astype(jnp.int32)


def _sc_scatter_body(x_hbm, p0_hbm, p1_hbm, xs_hbm, xbuf, idx0, idx1, sem):
    c = lax.axis_index("c")
    s = lax.axis_index("s")
    wid = s * 2 + c
    t0 = wid * TPW
    pltpu.sync_copy(x_hbm.at[pl.ds(t0, TPW)], xbuf)
    pltpu.sync_copy(p0_hbm.at[pl.ds(t0, TPW)], idx0)
    pltpu.sync_copy(p1_hbm.at[pl.ds(t0, TPW)], idx1)
    pltpu.async_copy(xbuf, xs_hbm.at[idx0], sem).wait()
    pltpu.async_copy(xbuf, xs_hbm.at[idx1], sem).wait()


def _ffn_body(te_ref, xs_ref, w1_ref, w3_ref, w2_ref, ys_ref):
    x = xs_ref[...].astype(jnp.bfloat16)
    h1 = lax.dot_general(x, w1_ref[0], (((1,), (1,)), ((), ())),
                         preferred_element_type=jnp.float32)
    h3 = lax.dot_general(x, w3_ref[0], (((1,), (1,)), ((), ())),
                         preferred_element_type=jnp.float32)
    act = (0.5 * h1) * (1.0 + lax.erf(h1 * 0.7071067811865476)) * h3
    ys_ref[...] = lax.dot_general(act.astype(jnp.bfloat16), w2_ref[0],
                                  (((1,), (1,)), ((), ())),
                                  preferred_element_type=jnp.float32)


def _sc_combine_body(ys_hbm, p0_hbm, p1_hbm, w0_hbm, w1_hbm, y_hbm,
                     wbuf0, wbuf1, idxa, idxb, buf0, buf1, ybuf, sem0, sem1):
    c = lax.axis_index("c")
    s = lax.axis_index("s")
    wid = s * 2 + c
    t0 = wid * TPW
    pltpu.sync_copy(w0_hbm.at[pl.ds(t0, TPW)], wbuf0)
    pltpu.sync_copy(w1_hbm.at[pl.ds(t0, TPW)], wbuf1)
    for ch in range(2):
        pltpu.sync_copy(p0_hbm.at[pl.ds(t0 + ch * 32, 32)], idxa)
        pltpu.sync_copy(p1_hbm.at[pl.ds(t0 + ch * 32, 32)], idxb)
        cpa = pltpu.async_copy(ys_hbm.at[idxa], buf0, sem0)
        cpb = pltpu.async_copy(ys_hbm.at[idxb], buf1, sem1)
        cpa.wait()
        cpb.wait()

        def row_body(i, carry, ch=ch):
            wa = wbuf0[ch * 32 + i]
            wb = wbuf1[ch * 32 + i]
            for cc in range(H // 16):
                sl = pl.ds(cc * 16, 16)
                ybuf[i, sl] = wa * buf0[i, sl] + wb * buf1[i, sl]
            return carry

        lax.fori_loop(0, 32, row_body, 0)
        pltpu.sync_copy(ybuf, y_hbm.at[pl.ds(t0 + ch * 32, 32)])


@jax.jit
def kernel(hidden_states, w_gate, w1, w3, w2):
    pos0, pos1, wv0, wv1, te_col = pl.pallas_call(
        _dispatch_body,
        out_shape=(jax.ShapeDtypeStruct((T, 1), jnp.int32),
                   jax.ShapeDtypeStruct((T, 1), jnp.int32),
                   jax.ShapeDtypeStruct((T, 16), jnp.float32),
                   jax.ShapeDtypeStruct((T, 16), jnp.float32),
                   jax.ShapeDtypeStruct((NT, 1), jnp.int32)),
        in_specs=[pl.BlockSpec((T, H), lambda: (0, 0)),
                  pl.BlockSpec((E, H), lambda: (0, 0))],
        out_specs=(pl.BlockSpec((T, 1), lambda: (0, 0)),
                   pl.BlockSpec((T, 1), lambda: (0, 0)),
                   pl.BlockSpec((T, 16), lambda: (0, 0)),
                   pl.BlockSpec((T, 16), lambda: (0, 0)),
                   pl.BlockSpec((NT, 1), lambda: (0, 0))),
    )(hidden_states, w_gate)
    p0f = pos0.reshape(T)
    p1f = pos1.reshape(T)

    mesh = plsc.VectorSubcoreMesh(core_axis_name="c", subcore_axis_name="s")
    xs = pl.kernel(
        _sc_scatter_body,
        out_type=jax.ShapeDtypeStruct((P, H), jnp.float32),
        mesh=mesh,
        scratch_types=[pltpu.VMEM((TPW, H), jnp.float32),
                       pltpu.VMEM((TPW,), jnp.int32),
                       pltpu.VMEM((TPW,), jnp.int32),
                       pltpu.SemaphoreType.DMA],
    )(hidden_states, p0f, p1f)

    w1_bf = w1.astype(jnp.bfloat16)
    w3_bf = w3.astype(jnp.bfloat16)
    w2_bf = w2.astype(jnp.bfloat16)
    te_arr = te_col.reshape(NT)

    grid_spec = pltpu.PrefetchScalarGridSpec(
        num_scalar_prefetch=1,
        grid=(NT,),
        in_specs=[
            pl.BlockSpec((TM, H), lambda j, te: (j, 0)),
            pl.BlockSpec((1, F, H), lambda j, te: (te[j], 0, 0)),
            pl.BlockSpec((1, F, H), lambda j, te: (te[j], 0, 0)),
            pl.BlockSpec((1, H, F), lambda j, te: (te[j], 0, 0)),
        ],
        out_specs=pl.BlockSpec((TM, H), lambda j, te: (j, 0)),
    )
    ys = pl.pallas_call(
        _ffn_body,
        grid_spec=grid_spec,
        out_shape=jax.ShapeDtypeStruct((P, H), jnp.float32),
        compiler_params=pltpu.CompilerParams(
            dimension_semantics=("arbitrary",)),
    )(te_arr, xs, w1_bf, w3_bf, w2_bf)

    y = pl.kernel(
        _sc_combine_body,
        out_type=jax.ShapeDtypeStruct((T, H), jnp.float32),
        mesh=mesh,
        scratch_types=[pltpu.VMEM((TPW, 16), jnp.float32),
                       pltpu.VMEM((TPW, 16), jnp.float32),
                       pltpu.VMEM((32,), jnp.int32),
                       pltpu.VMEM((32,), jnp.int32),
                       pltpu.VMEM((32, H), jnp.float32),
                       pltpu.VMEM((32, H), jnp.float32),
                       pltpu.VMEM((32, H), jnp.float32),
                       pltpu.SemaphoreType.DMA,
                       pltpu.SemaphoreType.DMA],
    )(ys, p0f, p1f, wv0, wv1)
    return y


# speedup vs baseline: 2.1863x; 1.3157x over previous
import functools

import jax
import jax.numpy as jnp
from jax import lax
from jax.experimental import pallas as pl
from jax.experimental.pallas import tpu as pltpu
from jax.experimental.pallas import tpu_sc as plsc

T = 2048
H = 1024
F = 2048
E = 8
SOFTCAP = 30.0

TM = 256
NT = 24
P = NT * TM
HI = H // 2
NW = 32
TPW = T // NW


def _dispatch_body(x_ref, wg_ref, p0_ref, p1_ref, w0_ref, w1_ref, te_ref):
    x = x_ref[...]
    logits = lax.dot_general(x, wg_ref[...], (((1,), (1,)), ((), ())),
                             preferred_element_type=jnp.float32)
    logits = SOFTCAP * jnp.tanh(logits / SOFTCAP)
    m = jnp.max(logits, axis=-1, keepdims=True)
    p = jnp.exp(logits - m)
    p = p / jnp.sum(p, axis=-1, keepdims=True)
    e_iota = lax.broadcasted_iota(jnp.int32, (T, E), 1)
    m1 = jnp.max(p, axis=-1, keepdims=True)
    i1 = jnp.min(jnp.where(p == m1, e_iota, E), axis=-1, keepdims=True)
    sel1 = e_iota == i1
    p2 = jnp.where(sel1, -jnp.inf, p)
    m2 = jnp.max(p2, axis=-1, keepdims=True)
    i2 = jnp.min(jnp.where(p2 == m2, e_iota, E), axis=-1, keepdims=True)
    sel2 = e_iota == i2
    s01 = (sel1 | sel2).astype(jnp.float32)

    nblk = T // TM
    lt = (lax.broadcasted_iota(jnp.int32, (TM, TM), 0) >
          lax.broadcasted_iota(jnp.int32, (TM, TM), 1)).astype(jnp.float32)
    ranks, bsums = [], []
    for b in range(nblk):
        sb = s01[b * TM:(b + 1) * TM, :]
        ranks.append(lax.dot_general(lt, sb, (((1,), (0,)), ((), ())),
                                     preferred_element_type=jnp.float32))
        bsums.append(jnp.sum(sb, axis=0, keepdims=True))
    bs = jnp.concatenate(bsums, axis=0)
    lt8 = (lax.broadcasted_iota(jnp.int32, (nblk, nblk), 0) >
           lax.broadcasted_iota(jnp.int32, (nblk, nblk), 1)).astype(jnp.float32)
    boffs = lax.dot_general(lt8, bs, (((1,), (0,)), ((), ())),
                            preferred_element_type=jnp.float32)
    rank = jnp.concatenate(
        [ranks[b] + boffs[b:b + 1, :] for b in range(nblk)], axis=0)
    counts = boffs[nblk - 1:nblk, :] + bs[nblk - 1:nblk, :]
    c_i = counts.astype(jnp.int32)
    nt_e = (c_i + (TM - 1)) // TM
    m8 = (lax.broadcasted_iota(jnp.int32, (E, E), 0) <
          lax.broadcasted_iota(jnp.int32, (E, E), 1)).astype(jnp.float32)
    offs_t = lax.dot_general(nt_e.astype(jnp.float32), m8,
                             (((1,), (0,)), ((), ())),
                             preferred_element_type=jnp.float32)
    offs_i = offs_t.astype(jnp.int32)
    dest = offs_i * TM + rank.astype(jnp.int32)
    p0_ref[...] = jnp.min(jnp.where(sel1, dest, P), axis=-1, keepdims=True)
    p1_ref[...] = jnp.min(jnp.where(sel2, dest, P), axis=-1, keepdims=True)
    w0_ref[...] = jnp.broadcast_to(m1, (T, 16))
    w1_ref[...] = jnp.broadcast_to(m2, (T, 16))
    jt = lax.broadcasted_iota(jnp.int32, (NT, E), 0)
    step = (jt >= offs_i).astype(jnp.float32)
    te = lax.dot_general(step, jnp.ones((E, 1), jnp.float32),
                         (((1,), (0,)), ((), ())),
                         preferred_element_type=jnp.float32) - 1.0
    te_ref[...] = te.astype(jnp.int32)


def _sc_scatter_body(x_hbm, p0_hbm, p1_hbm, xs_hbm, xbuf, idx0, idx1, sem):
    c = lax.axis_index("c")
    s = lax.axis_index("s")
    wid = s * 2 + c
    t0 = wid * TPW
    pltpu.sync_copy(x_hbm.at[pl.ds(t0, TPW)], xbuf)
    pltpu.sync_copy(p0_hbm.at[pl.ds(t0, TPW)], idx0)
    pltpu.sync_copy(p1_hbm.at[pl.ds(t0, TPW)], idx1)
    pltpu.async_copy(xbuf, xs_hbm.at[idx0], sem).wait()
    pltpu.async_copy(xbuf, xs_hbm.at[idx1], sem).wait()


def _ffn_body(te_ref, xs_ref, w1_ref, w3_ref, w2_ref, ys_ref):
    x = xs_ref[...]
    h1 = lax.dot_general(x, w1_ref[0], (((1,), (1,)), ((), ())),
                         preferred_element_type=jnp.float32)
    h3 = lax.dot_general(x, w3_ref[0], (((1,), (1,)), ((), ())),
                         preferred_element_type=jnp.float32)
    act = (0.5 * h1) * (1.0 + lax.erf(h1 * 0.7071067811865476)) * h3
    ys_ref[...] = lax.dot_general(act, w2_ref[0],
                                  (((1,), (1,)), ((), ())),
                                  preferred_element_type=jnp.float32)


def _sc_combine_body(ys_hbm, p0_hbm, p1_hbm, w0_hbm, w1_hbm, y_hbm,
                     wbuf0, wbuf1, idxa, idxb, buf0, buf1, ybuf, sem0, sem1):
    c = lax.axis_index("c")
    s = lax.axis_index("s")
    wid = s * 2 + c
    t0 = wid * TPW
    pltpu.sync_copy(w0_hbm.at[pl.ds(t0, TPW)], wbuf0)
    pltpu.sync_copy(w1_hbm.at[pl.ds(t0, TPW)], wbuf1)
    for ch in range(2):
        pltpu.sync_copy(p0_hbm.at[pl.ds(t0 + ch * 32, 32)], idxa)
        pltpu.sync_copy(p1_hbm.at[pl.ds(t0 + ch * 32, 32)], idxb)
        cpa = pltpu.async_copy(ys_hbm.at[idxa], buf0, sem0)
        cpb = pltpu.async_copy(ys_hbm.at[idxb], buf1, sem1)
        cpa.wait()
        cpb.wait()

        def row_body(i, carry, ch=ch):
            wa = wbuf0[ch * 32 + i]
            wb = wbuf1[ch * 32 + i]
            for cc in range(H // 16):
                sl = pl.ds(cc * 16, 16)
                ybuf[i, sl] = wa * buf0[i, sl] + wb * buf1[i, sl]
            return carry

        lax.fori_loop(0, 32, row_body, 0)
        pltpu.sync_copy(ybuf, y_hbm.at[pl.ds(t0 + ch * 32, 32)])


@jax.jit
def kernel(hidden_states, w_gate, w1, w3, w2):
    pos0, pos1, wv0, wv1, te_col = pl.pallas_call(
        _dispatch_body,
        out_shape=(jax.ShapeDtypeStruct((T, 1), jnp.int32),
                   jax.ShapeDtypeStruct((T, 1), jnp.int32),
                   jax.ShapeDtypeStruct((T, 16), jnp.float32),
                   jax.ShapeDtypeStruct((T, 16), jnp.float32),
                   jax.ShapeDtypeStruct((NT, 1), jnp.int32)),
        in_specs=[pl.BlockSpec((T, H), lambda: (0, 0)),
                  pl.BlockSpec((E, H), lambda: (0, 0))],
        out_specs=(pl.BlockSpec((T, 1), lambda: (0, 0)),
                   pl.BlockSpec((T, 1), lambda: (0, 0)),
                   pl.BlockSpec((T, 16), lambda: (0, 0)),
                   pl.BlockSpec((T, 16), lambda: (0, 0)),
                   pl.BlockSpec((NT, 1), lambda: (0, 0))),
    )(hidden_states, w_gate)
    p0f = pos0.reshape(T)
    p1f = pos1.reshape(T)

    mesh = plsc.VectorSubcoreMesh(core_axis_name="c", subcore_axis_name="s")
    xs = pl.kernel(
        _sc_scatter_body,
        out_type=jax.ShapeDtypeStruct((P, H), jnp.float32),
        mesh=mesh,
        scratch_types=[pltpu.VMEM((TPW, H), jnp.float32),
                       pltpu.VMEM((TPW,), jnp.int32),
                       pltpu.VMEM((TPW,), jnp.int32),
                       pltpu.SemaphoreType.DMA],
    )(hidden_states, p0f, p1f)

    te_arr = te_col.reshape(NT)

    grid_spec = pltpu.PrefetchScalarGridSpec(
        num_scalar_prefetch=1,
        grid=(NT,),
        in_specs=[
            pl.BlockSpec((TM, H), lambda j, te: (j, 0)),
            pl.BlockSpec((1, F, H), lambda j, te: (te[j], 0, 0)),
            pl.BlockSpec((1, F, H), lambda j, te: (te[j], 0, 0)),
            pl.BlockSpec((1, H, F), lambda j, te: (te[j], 0, 0)),
        ],
        out_specs=pl.BlockSpec((TM, H), lambda j, te: (j, 0)),
    )
    ys = pl.pallas_call(
        _ffn_body,
        grid_spec=grid_spec,
        out_shape=jax.ShapeDtypeStruct((P, H), jnp.float32),
        compiler_params=pltpu.CompilerParams(
            dimension_semantics=("arbitrary",)),
    )(te_arr, xs, w1, w3, w2)

    y = pl.kernel(
        _sc_combine_body,
        out_type=jax.ShapeDtypeStruct((T, H), jnp.float32),
        mesh=mesh,
        scratch_types=[pltpu.VMEM((TPW, 16), jnp.float32),
                       pltpu.VMEM((TPW, 16), jnp.float32),
                       pltpu.VMEM((32,), jnp.int32),
                       pltpu.VMEM((32,), jnp.int32),
                       pltpu.VMEM((32, H), jnp.float32),
                       pltpu.VMEM((32, H), jnp.float32),
                       pltpu.VMEM((32, H), jnp.float32),
                       pltpu.SemaphoreType.DMA,
                       pltpu.SemaphoreType.DMA],
    )(ys, p0f, p1f, wv0, wv1)
    return y


# speedup vs baseline: 2.3844x; 1.0906x over previous
import functools

import jax
import jax.numpy as jnp
from jax import lax
from jax.experimental import pallas as pl
from jax.experimental.pallas import tpu as pltpu
from jax.experimental.pallas import tpu_sc as plsc

T = 2048
H = 1024
F = 2048
E = 8
SOFTCAP = 30.0

TM = 256
NT = 24
P = NT * TM
HI = H // 2
NW = 32
TPW = T // NW


def _dispatch_body(x_ref, wg_ref, p0_ref, p1_ref, w0_ref, w1_ref, te_ref,
                   tv_ref):
    x = x_ref[...]
    logits = lax.dot_general(x, wg_ref[...], (((1,), (1,)), ((), ())),
                             preferred_element_type=jnp.float32)
    logits = SOFTCAP * jnp.tanh(logits / SOFTCAP)
    m = jnp.max(logits, axis=-1, keepdims=True)
    p = jnp.exp(logits - m)
    p = p / jnp.sum(p, axis=-1, keepdims=True)
    e_iota = lax.broadcasted_iota(jnp.int32, (T, E), 1)
    m1 = jnp.max(p, axis=-1, keepdims=True)
    i1 = jnp.min(jnp.where(p == m1, e_iota, E), axis=-1, keepdims=True)
    sel1 = e_iota == i1
    p2 = jnp.where(sel1, -jnp.inf, p)
    m2 = jnp.max(p2, axis=-1, keepdims=True)
    i2 = jnp.min(jnp.where(p2 == m2, e_iota, E), axis=-1, keepdims=True)
    sel2 = e_iota == i2
    s01 = (sel1 | sel2).astype(jnp.float32)

    nblk = T // TM
    lt = (lax.broadcasted_iota(jnp.int32, (TM, TM), 0) >
          lax.broadcasted_iota(jnp.int32, (TM, TM), 1)).astype(jnp.float32)
    ranks, bsums = [], []
    for b in range(nblk):
        sb = s01[b * TM:(b + 1) * TM, :]
        ranks.append(lax.dot_general(lt, sb, (((1,), (0,)), ((), ())),
                                     preferred_element_type=jnp.float32))
        bsums.append(jnp.sum(sb, axis=0, keepdims=True))
    bs = jnp.concatenate(bsums, axis=0)
    lt8 = (lax.broadcasted_iota(jnp.int32, (nblk, nblk), 0) >
           lax.broadcasted_iota(jnp.int32, (nblk, nblk), 1)).astype(jnp.float32)
    boffs = lax.dot_general(lt8, bs, (((1,), (0,)), ((), ())),
                            preferred_element_type=jnp.float32)
    rank = jnp.concatenate(
        [ranks[b] + boffs[b:b + 1, :] for b in range(nblk)], axis=0)
    counts = boffs[nblk - 1:nblk, :] + bs[nblk - 1:nblk, :]
    c_i = counts.astype(jnp.int32)
    nt_e = (c_i + (TM - 1)) // TM
    m8 = (lax.broadcasted_iota(jnp.int32, (E, E), 0) <
          lax.broadcasted_iota(jnp.int32, (E, E), 1)).astype(jnp.float32)
    offs_t = lax.dot_general(nt_e.astype(jnp.float32), m8,
                             (((1,), (0,)), ((), ())),
                             preferred_element_type=jnp.float32)
    offs_i = offs_t.astype(jnp.int32)
    dest = offs_i * TM + rank.astype(jnp.int32)
    p0_ref[...] = jnp.min(jnp.where(sel1, dest, P), axis=-1, keepdims=True)
    p1_ref[...] = jnp.min(jnp.where(sel2, dest, P), axis=-1, keepdims=True)
    w0_ref[...] = jnp.broadcast_to(m1, (T, 16))
    w1_ref[...] = jnp.broadcast_to(m2, (T, 16))
    jt = lax.broadcasted_iota(jnp.int32, (NT, E), 0)
    step = (jt >= offs_i).astype(jnp.float32)
    te = lax.dot_general(step, jnp.ones((E, 1), jnp.float32),
                         (((1,), (0,)), ((), ())),
                         preferred_element_type=jnp.float32) - 1.0
    te_ref[...] = te.astype(jnp.int32)
    onehot_last = (lax.broadcasted_iota(jnp.int32, (E, 1), 0) == (E - 1)
                   ).astype(jnp.float32)
    total = lax.dot_general((offs_t + nt_e.astype(jnp.float32)), onehot_last,
                            (((1,), (0,)), ((), ())),
                            preferred_element_type=jnp.float32)
    jt1 = lax.broadcasted_iota(jnp.int32, (NT, 1), 0)
    tv_ref[...] = (jt1 < total.astype(jnp.int32)).astype(jnp.int32)


def _sc_scatter_body(x_hbm, p0_hbm, p1_hbm, xs_hbm, xbuf, idx0, idx1,
                     sem0, sem1, semx):
    c = lax.axis_index("c")
    s = lax.axis_index("s")
    wid = s * 2 + c
    t0 = wid * TPW
    cpx = pltpu.async_copy(x_hbm.at[pl.ds(t0, TPW)], xbuf, semx)
    cp0 = pltpu.async_copy(p0_hbm.at[pl.ds(t0, TPW)], idx0, sem0)
    cp1 = pltpu.async_copy(p1_hbm.at[pl.ds(t0, TPW)], idx1, sem1)
    cpx.wait()
    cp0.wait()
    cp1.wait()
    cs0 = pltpu.async_copy(xbuf, xs_hbm.at[idx0], sem0)
    cs1 = pltpu.async_copy(xbuf, xs_hbm.at[idx1], sem1)
    cs0.wait()
    cs1.wait()


def _ffn_body(te_ref, tv_ref, xs_ref, w1_ref, w3_ref, w2_ref, ys_ref):
    @pl.when(tv_ref[pl.program_id(0)] != 0)
    def _live():
        x = xs_ref[...]
        h1 = lax.dot_general(x, w1_ref[0], (((1,), (1,)), ((), ())),
                             preferred_element_type=jnp.float32)
        h3 = lax.dot_general(x, w3_ref[0], (((1,), (1,)), ((), ())),
                             preferred_element_type=jnp.float32)
        act = (0.5 * h1) * (1.0 + lax.erf(h1 * 0.7071067811865476)) * h3
        ys_ref[...] = lax.dot_general(act, w2_ref[0],
                                      (((1,), (1,)), ((), ())),
                                      preferred_element_type=jnp.float32)


_CCH = 16
_NCH = TPW // _CCH


def _sc_combine_body(ys_hbm, p0_hbm, p1_hbm, w0_hbm, w1_hbm, y_hbm,
                     wbuf0, wbuf1, idxa0, idxb0, idxa1, idxb1,
                     bufa0, bufb0, bufa1, bufb1, ybuf,
                     sa0, sb0, sa1, sb1):
    c = lax.axis_index("c")
    s = lax.axis_index("s")
    wid = s * 2 + c
    t0 = wid * TPW
    pltpu.sync_copy(w0_hbm.at[pl.ds(t0, TPW)], wbuf0)
    pltpu.sync_copy(w1_hbm.at[pl.ds(t0, TPW)], wbuf1)
    bufs = ((bufa0, bufb0), (bufa1, bufb1))
    sems = ((sa0, sb0), (sa1, sb1))
    idxs = ((idxa0, idxb0), (idxa1, idxb1))

    def _start(ch):
        ba, bb = bufs[ch % 2]
        ma, mb = sems[ch % 2]
        ia, ib = idxs[ch % 2]
        pltpu.sync_copy(p0_hbm.at[pl.ds(t0 + ch * _CCH, _CCH)], ia)
        pltpu.sync_copy(p1_hbm.at[pl.ds(t0 + ch * _CCH, _CCH)], ib)
        cpa = pltpu.async_copy(ys_hbm.at[ia], ba, ma)
        cpb = pltpu.async_copy(ys_hbm.at[ib], bb, mb)
        return cpa, cpb

    inflight = {0: _start(0)}
    for ch in range(_NCH):
        if ch + 1 < _NCH:
            inflight[ch + 1] = _start(ch + 1)
        cpa, cpb = inflight.pop(ch)
        cpa.wait()
        cpb.wait()
        ba, bb = bufs[ch % 2]

        def row_body(i, carry, ch=ch, ba=ba, bb=bb):
            wa = wbuf0[ch * _CCH + i]
            wb = wbuf1[ch * _CCH + i]
            for cc in range(H // 16):
                sl = pl.ds(cc * 16, 16)
                ybuf[i, sl] = wa * ba[i, sl] + wb * bb[i, sl]
            return carry

        lax.fori_loop(0, _CCH, row_body, 0)
        pltpu.sync_copy(ybuf, y_hbm.at[pl.ds(t0 + ch * _CCH, _CCH)])


@jax.jit
def kernel(hidden_states, w_gate, w1, w3, w2):
    pos0, pos1, wv0, wv1, te_col, tv_col = pl.pallas_call(
        _dispatch_body,
        out_shape=(jax.ShapeDtypeStruct((T, 1), jnp.int32),
                   jax.ShapeDtypeStruct((T, 1), jnp.int32),
                   jax.ShapeDtypeStruct((T, 16), jnp.float32),
                   jax.ShapeDtypeStruct((T, 16), jnp.float32),
                   jax.ShapeDtypeStruct((NT, 1), jnp.int32),
                   jax.ShapeDtypeStruct((NT, 1), jnp.int32)),
        in_specs=[pl.BlockSpec((T, H), lambda: (0, 0)),
                  pl.BlockSpec((E, H), lambda: (0, 0))],
        out_specs=(pl.BlockSpec((T, 1), lambda: (0, 0)),
                   pl.BlockSpec((T, 1), lambda: (0, 0)),
                   pl.BlockSpec((T, 16), lambda: (0, 0)),
                   pl.BlockSpec((T, 16), lambda: (0, 0)),
                   pl.BlockSpec((NT, 1), lambda: (0, 0)),
                   pl.BlockSpec((NT, 1), lambda: (0, 0))),
    )(hidden_states, w_gate)
    p0f = pos0.reshape(T)
    p1f = pos1.reshape(T)

    mesh = plsc.VectorSubcoreMesh(core_axis_name="c", subcore_axis_name="s")
    xs = pl.kernel(
        _sc_scatter_body,
        out_type=jax.ShapeDtypeStruct((P, H), jnp.float32),
        mesh=mesh,
        scratch_types=[pltpu.VMEM((TPW, H), jnp.float32),
                       pltpu.VMEM((TPW,), jnp.int32),
                       pltpu.VMEM((TPW,), jnp.int32),
                       pltpu.SemaphoreType.DMA,
                       pltpu.SemaphoreType.DMA,
                       pltpu.SemaphoreType.DMA],
    )(hidden_states, p0f, p1f)

    te_arr = te_col.reshape(NT)
    tv_arr = tv_col.reshape(NT)

    grid_spec = pltpu.PrefetchScalarGridSpec(
        num_scalar_prefetch=2,
        grid=(NT,),
        in_specs=[
            pl.BlockSpec((TM, H), lambda j, te, tv: (j, 0)),
            pl.BlockSpec((1, F, H), lambda j, te, tv: (te[j], 0, 0)),
            pl.BlockSpec((1, F, H), lambda j, te, tv: (te[j], 0, 0)),
            pl.BlockSpec((1, H, F), lambda j, te, tv: (te[j], 0, 0)),
        ],
        out_specs=pl.BlockSpec((TM, H), lambda j, te, tv: (j, 0)),
    )
    ys = pl.pallas_call(
        _ffn_body,
        grid_spec=grid_spec,
        out_shape=jax.ShapeDtypeStruct((P, H), jnp.float32),
        compiler_params=pltpu.CompilerParams(
            dimension_semantics=("arbitrary",)),
    )(te_arr, tv_arr, xs, w1, w3, w2)

    y = pl.kernel(
        _sc_combine_body,
        out_type=jax.ShapeDtypeStruct((T, H), jnp.float32),
        mesh=mesh,
        scratch_types=[pltpu.VMEM((TPW, 16), jnp.float32),
                       pltpu.VMEM((TPW, 16), jnp.float32),
                       pltpu.VMEM((_CCH,), jnp.int32),
                       pltpu.VMEM((_CCH,), jnp.int32),
                       pltpu.VMEM((_CCH,), jnp.int32),
                       pltpu.VMEM((_CCH,), jnp.int32),
                       pltpu.VMEM((_CCH, H), jnp.float32),
                       pltpu.VMEM((_CCH, H), jnp.float32),
                       pltpu.VMEM((_CCH, H), jnp.float32),
                       pltpu.VMEM((_CCH, H), jnp.float32),
                       pltpu.VMEM((_CCH, H), jnp.float32),
                       pltpu.SemaphoreType.DMA,
                       pltpu.SemaphoreType.DMA,
                       pltpu.SemaphoreType.DMA,
                       pltpu.SemaphoreType.DMA],
    )(ys, p0f, p1f, wv0, wv1)
    return y


# speedup vs baseline: 2.3892x; 1.0020x over previous
import jax
import jax.numpy as jnp
from jax import lax
from jax.experimental import pallas as pl
from jax.experimental.pallas import tpu as pltpu
from jax.experimental.pallas import tpu_sc as plsc

T = 2048
H = 1024
F = 2048
E = 8
SOFTCAP = 30.0

TM = 256
NT = 24
P = NT * TM
NW = 32
TPW = T // NW


def _dispatch_body(x_ref, wg_ref, p0_ref, p1_ref, w0_ref, w1_ref, te_ref,
                   tv_ref):
    x = x_ref[...]
    logits = lax.dot_general(x, wg_ref[...], (((1,), (1,)), ((), ())),
                             preferred_element_type=jnp.float32)
    logits = SOFTCAP * jnp.tanh(logits / SOFTCAP)
    m = jnp.max(logits, axis=-1, keepdims=True)
    p = jnp.exp(logits - m)
    p = p / jnp.sum(p, axis=-1, keepdims=True)
    e_iota = lax.broadcasted_iota(jnp.int32, (T, E), 1)
    m1 = jnp.max(p, axis=-1, keepdims=True)
    i1 = jnp.min(jnp.where(p == m1, e_iota, E), axis=-1, keepdims=True)
    sel1 = e_iota == i1
    p2 = jnp.where(sel1, -jnp.inf, p)
    m2 = jnp.max(p2, axis=-1, keepdims=True)
    i2 = jnp.min(jnp.where(p2 == m2, e_iota, E), axis=-1, keepdims=True)
    sel2 = e_iota == i2
    s01 = (sel1 | sel2).astype(jnp.float32)

    nblk = T // TM
    lt = (lax.broadcasted_iota(jnp.int32, (TM, TM), 0) >
          lax.broadcasted_iota(jnp.int32, (TM, TM), 1)).astype(jnp.float32)
    ranks, bsums = [], []
    for b in range(nblk):
        sb = s01[b * TM:(b + 1) * TM, :]
        ranks.append(lax.dot_general(lt, sb, (((1,), (0,)), ((), ())),
                                     preferred_element_type=jnp.float32))
        bsums.append(jnp.sum(sb, axis=0, keepdims=True))
    bs = jnp.concatenate(bsums, axis=0)
    lt8 = (lax.broadcasted_iota(jnp.int32, (nblk, nblk), 0) >
           lax.broadcasted_iota(jnp.int32, (nblk, nblk), 1)).astype(jnp.float32)
    boffs = lax.dot_general(lt8, bs, (((1,), (0,)), ((), ())),
                            preferred_element_type=jnp.float32)
    rank = jnp.concatenate(
        [ranks[b] + boffs[b:b + 1, :] for b in range(nblk)], axis=0)
    counts = boffs[nblk - 1:nblk, :] + bs[nblk - 1:nblk, :]
    c_i = counts.astype(jnp.int32)
    nt_e = (c_i + (TM - 1)) // TM
    m8 = (lax.broadcasted_iota(jnp.int32, (E, E), 0) <
          lax.broadcasted_iota(jnp.int32, (E, E), 1)).astype(jnp.float32)
    offs_t = lax.dot_general(nt_e.astype(jnp.float32), m8,
                             (((1,), (0,)), ((), ())),
                             preferred_element_type=jnp.float32)
    offs_i = offs_t.astype(jnp.int32)
    dest = offs_i * TM + rank.astype(jnp.int32)
    p0_ref[...] = jnp.min(jnp.where(sel1, dest, P), axis=-1, keepdims=True)
    p1_ref[...] = jnp.min(jnp.where(sel2, dest, P), axis=-1, keepdims=True)
    w0_ref[...] = jnp.broadcast_to(m1, (T, 16))
    w1_ref[...] = jnp.broadcast_to(m2, (T, 16))
    jt = lax.broadcasted_iota(jnp.int32, (NT, E), 0)
    step = (jt >= offs_i).astype(jnp.float32)
    te = lax.dot_general(step, jnp.ones((E, 1), jnp.float32),
                         (((1,), (0,)), ((), ())),
                         preferred_element_type=jnp.float32) - 1.0
    te_ref[...] = te.astype(jnp.int32)
    onehot_last = (lax.broadcasted_iota(jnp.int32, (E, 1), 0) == (E - 1)
                   ).astype(jnp.float32)
    total = lax.dot_general((offs_t + nt_e.astype(jnp.float32)), onehot_last,
                            (((1,), (0,)), ((), ())),
                            preferred_element_type=jnp.float32)
    jt1 = lax.broadcasted_iota(jnp.int32, (NT, 1), 0)
    tv_ref[...] = (jt1 < total.astype(jnp.int32)).astype(jnp.int32)


def _sc_scatter_body(x_hbm, p0_hbm, p1_hbm, xs_hbm, xbuf, idx0, idx1,
                     sem0, sem1, semx):
    c = lax.axis_index("c")
    s = lax.axis_index("s")
    wid = s * 2 + c
    t0 = wid * TPW
    cpx = pltpu.async_copy(x_hbm.at[pl.ds(t0, TPW)], xbuf, semx)
    cp0 = pltpu.async_copy(p0_hbm.at[pl.ds(t0, TPW)], idx0, sem0)
    cp1 = pltpu.async_copy(p1_hbm.at[pl.ds(t0, TPW)], idx1, sem1)
    cpx.wait()
    cp0.wait()
    cp1.wait()
    cs0 = pltpu.async_copy(xbuf, xs_hbm.at[idx0], sem0)
    cs1 = pltpu.async_copy(xbuf, xs_hbm.at[idx1], sem1)
    cs0.wait()
    cs1.wait()


def _ffn_body(te_ref, tv_ref, xs_ref, w1_ref, w3_ref, w2_ref, ys_ref):
    @pl.when(tv_ref[pl.program_id(0)] != 0)
    def _live():
        x = xs_ref[...]
        h1 = lax.dot_general(x, w1_ref[0], (((1,), (1,)), ((), ())),
                             preferred_element_type=jnp.float32)
        h3 = lax.dot_general(x, w3_ref[0], (((1,), (1,)), ((), ())),
                             preferred_element_type=jnp.float32)
        act = (0.5 * h1) * (1.0 + lax.erf(h1 * 0.7071067811865476)) * h3
        ys_ref[...] = lax.dot_general(act, w2_ref[0],
                                      (((1,), (1,)), ((), ())),
                                      preferred_element_type=jnp.float32)


_CCH = 16
_NCH = TPW // _CCH


def _sc_combine_body(ys_hbm, p0_hbm, p1_hbm, w0_hbm, w1_hbm, y_hbm,
                     wbuf0, wbuf1, idxa0, idxb0, idxa1, idxb1,
                     bufa0, bufb0, bufa1, bufb1, ybuf,
                     sa0, sb0, sa1, sb1):
    c = lax.axis_index("c")
    s = lax.axis_index("s")
    wid = s * 2 + c
    t0 = wid * TPW
    pltpu.sync_copy(w0_hbm.at[pl.ds(t0, TPW)], wbuf0)
    pltpu.sync_copy(w1_hbm.at[pl.ds(t0, TPW)], wbuf1)
    bufs = ((bufa0, bufb0), (bufa1, bufb1))
    sems = ((sa0, sb0), (sa1, sb1))
    idxs = ((idxa0, idxb0), (idxa1, idxb1))

    def _start(ch):
        ba, bb = bufs[ch % 2]
        ma, mb = sems[ch % 2]
        ia, ib = idxs[ch % 2]
        pltpu.sync_copy(p0_hbm.at[pl.ds(t0 + ch * _CCH, _CCH)], ia)
        pltpu.sync_copy(p1_hbm.at[pl.ds(t0 + ch * _CCH, _CCH)], ib)
        cpa = pltpu.async_copy(ys_hbm.at[ia], ba, ma)
        cpb = pltpu.async_copy(ys_hbm.at[ib], bb, mb)
        return cpa, cpb

    inflight = {0: _start(0)}
    for ch in range(_NCH):
        if ch + 1 < _NCH:
            inflight[ch + 1] = _start(ch + 1)
        cpa, cpb = inflight.pop(ch)
        cpa.wait()
        cpb.wait()
        ba, bb = bufs[ch % 2]

        def row_body(i, carry, ch=ch, ba=ba, bb=bb):
            wa = wbuf0[ch * _CCH + i]
            wb = wbuf1[ch * _CCH + i]
            for cc in range(H // 16):
                sl = pl.ds(cc * 16, 16)
                ybuf[i, sl] = wa * ba[i, sl] + wb * bb[i, sl]
            return carry

        lax.fori_loop(0, _CCH, row_body, 0)
        pltpu.sync_copy(ybuf, y_hbm.at[pl.ds(t0 + ch * _CCH, _CCH)])


@jax.jit
def kernel(hidden_states, w_gate, w1, w3, w2):
    pos0, pos1, wv0, wv1, te_col, tv_col = pl.pallas_call(
        _dispatch_body,
        out_shape=(jax.ShapeDtypeStruct((T, 1), jnp.int32),
                   jax.ShapeDtypeStruct((T, 1), jnp.int32),
                   jax.ShapeDtypeStruct((T, 16), jnp.float32),
                   jax.ShapeDtypeStruct((T, 16), jnp.float32),
                   jax.ShapeDtypeStruct((NT, 1), jnp.int32),
                   jax.ShapeDtypeStruct((NT, 1), jnp.int32)),
        in_specs=[pl.BlockSpec((T, H), lambda: (0, 0)),
                  pl.BlockSpec((E, H), lambda: (0, 0))],
        out_specs=(pl.BlockSpec((T, 1), lambda: (0, 0)),
                   pl.BlockSpec((T, 1), lambda: (0, 0)),
                   pl.BlockSpec((T, 16), lambda: (0, 0)),
                   pl.BlockSpec((T, 16), lambda: (0, 0)),
                   pl.BlockSpec((NT, 1), lambda: (0, 0)),
                   pl.BlockSpec((NT, 1), lambda: (0, 0))),
    )(hidden_states, w_gate)
    p0f = pos0.reshape(T)
    p1f = pos1.reshape(T)

    mesh = plsc.VectorSubcoreMesh(core_axis_name="c", subcore_axis_name="s")
    xs = pl.kernel(
        _sc_scatter_body,
        out_type=jax.ShapeDtypeStruct((P, H), jnp.float32),
        mesh=mesh,
        scratch_types=[pltpu.VMEM((TPW, H), jnp.float32),
                       pltpu.VMEM((TPW,), jnp.int32),
                       pltpu.VMEM((TPW,), jnp.int32),
                       pltpu.SemaphoreType.DMA,
                       pltpu.SemaphoreType.DMA,
                       pltpu.SemaphoreType.DMA],
    )(hidden_states, p0f, p1f)

    te_arr = te_col.reshape(NT)
    tv_arr = tv_col.reshape(NT)

    grid_spec = pltpu.PrefetchScalarGridSpec(
        num_scalar_prefetch=2,
        grid=(NT,),
        in_specs=[
            pl.BlockSpec((TM, H), lambda j, te, tv: (j, 0)),
            pl.BlockSpec((1, F, H), lambda j, te, tv: (te[j], 0, 0)),
            pl.BlockSpec((1, F, H), lambda j, te, tv: (te[j], 0, 0)),
            pl.BlockSpec((1, H, F), lambda j, te, tv: (te[j], 0, 0)),
        ],
        out_specs=pl.BlockSpec((TM, H), lambda j, te, tv: (j, 0)),
    )
    ys = pl.pallas_call(
        _ffn_body,
        grid_spec=grid_spec,
        out_shape=jax.ShapeDtypeStruct((P, H), jnp.float32),
        compiler_params=pltpu.CompilerParams(
            dimension_semantics=("arbitrary",)),
    )(te_arr, tv_arr, xs, w1, w3, w2)

    y = pl.kernel(
        _sc_combine_body,
        out_type=jax.ShapeDtypeStruct((T, H), jnp.float32),
        mesh=mesh,
        scratch_types=[pltpu.VMEM((TPW, 16), jnp.float32),
                       pltpu.VMEM((TPW, 16), jnp.float32),
                       pltpu.VMEM((_CCH,), jnp.int32),
                       pltpu.VMEM((_CCH,), jnp.int32),
                       pltpu.VMEM((_CCH,), jnp.int32),
                       pltpu.VMEM((_CCH,), jnp.int32),
                       pltpu.VMEM((_CCH, H), jnp.float32),
                       pltpu.VMEM((_CCH, H), jnp.float32),
                       pltpu.VMEM((_CCH, H), jnp.float32),
                       pltpu.VMEM((_CCH, H), jnp.float32),
                       pltpu.VMEM((_CCH, H), jnp.float32),
                       pltpu.SemaphoreType.DMA,
                       pltpu.SemaphoreType.DMA,
                       pltpu.SemaphoreType.DMA,
                       pltpu.SemaphoreType.DMA],
    )(ys, p0f, p1f, wv0, wv1)
    return y
